# baseline (speedup 1.0000x reference)
import functools

import jax
import jax.numpy as jnp
from jax import lax
from jax.experimental import pallas as pl
from jax.experimental.pallas import tpu as pltpu
from jax.experimental.pallas import tpu_sc as plsc

N_NODES = 10000
IN_FEATS = 128
OUT_FEATS = 128
N_EDGES = 320000

NC = 2
NS = 16
CH = 40
EPW = N_EDGES // (NC * NS)
NCHUNK = EPW // CH
NPAD = 10240
ROWS_PER_SUB = NPAD // NS
ZROWS = 32
NB = 5


def _poly_body(x_ref, c_ref, e_ref, y_ref, s_ref, d_ref):
    x = x_ref[...]
    dn = (((1,), (0,)), ((), ()))
    y = jnp.sum(c_ref[0], axis=0)[None, :]
    y = y + lax.dot_general(x, c_ref[1], dn, preferred_element_type=jnp.float32)
    x2 = x * x
    y = y + lax.dot_general(x2, c_ref[2], dn, preferred_element_type=jnp.float32)
    y = y + lax.dot_general(x2 * x, c_ref[3], dn, preferred_element_type=jnp.float32)
    y_ref[...] = y
    e = e_ref[...]
    eb = e.shape[1]
    i = pl.program_id(0)
    s_ref[pl.ds(i * eb, eb)] = e[0]
    d_ref[pl.ds(i * eb, eb)] = e[1]


def _tc_poly(x, cw, edge_index):
    blk = 2000
    grid = N_NODES // blk
    eb = N_EDGES // grid
    return pl.pallas_call(
        _poly_body,
        grid=(grid,),
        in_specs=[
            pl.BlockSpec((blk, IN_FEATS), lambda i: (i, 0)),
            pl.BlockSpec((4, IN_FEATS, OUT_FEATS), lambda i: (0, 0, 0)),
            pl.BlockSpec((2, eb), lambda i: (0, i)),
        ],
        out_specs=[
            pl.BlockSpec((blk, OUT_FEATS), lambda i: (i, 0)),
            pl.BlockSpec((N_EDGES,), lambda i: (0,)),
            pl.BlockSpec((N_EDGES,), lambda i: (0,)),
        ],
        out_shape=[
            jax.ShapeDtypeStruct((N_NODES, OUT_FEATS), jnp.float32),
            jax.ShapeDtypeStruct((N_EDGES,), jnp.int32),
            jax.ShapeDtypeStruct((N_EDGES,), jnp.int32),
        ],
    )(x, cw, edge_index)


def _sc_body(y_hbm, src_hbm, dst_hbm, out_hbm,
             sidxr, didxr, rows0, rows1, rows2, rows3, rows4, zbuf, acc, sems):
    c = lax.axis_index("c")
    s = lax.axis_index("s")
    w = c * NS + s
    base = w * EPW
    r0 = s * ROWS_PER_SUB
    rows = (rows0, rows1, rows2, rows3, rows4)
    gsem = tuple(sems.at[t] for t in range(NB))
    ssem = tuple(sems.at[NB + t] for t in range(NB))
    isem = tuple(sems.at[2 * NB + t] for t in range(NB))
    idsem = tuple(sems.at[3 * NB + t] for t in range(NB))
    zsem = sems.at[4 * NB]

    def fire_idx(k, slot):
        off = base + k * CH
        pltpu.async_copy(src_hbm.at[pl.ds(off, CH)], sidxr.at[slot],
                         isem[slot])
        pltpu.async_copy(dst_hbm.at[pl.ds(off, CH)], didxr.at[slot],
                         idsem[slot])

    def fire_gather(slot):
        pltpu.make_async_copy(src_hbm.at[pl.ds(0, CH)], sidxr.at[slot],
                              isem[slot]).wait()
        pltpu.async_copy(y_hbm.at[sidxr.at[slot]], rows[slot], gsem[slot])

    for t in range(NB):
        fire_idx(t, t)
    for t in range(3):
        fire_gather(t)

    def zrow(r, carry):
        for q in range(OUT_FEATS // 16):
            zbuf[r, pl.ds(q * 16, 16)] = jnp.zeros((16,), jnp.float32)
        return carry
    lax.fori_loop(0, ZROWS, zrow, 0)
    for t in range(ROWS_PER_SUB // ZROWS):
        pltpu.async_copy(zbuf, acc.at[pl.ds(r0 + t * ZROWS, ZROWS)], zsem)
    for t in range(ROWS_PER_SUB // ZROWS):
        pltpu.make_async_copy(zbuf, acc.at[pl.ds(r0, ZROWS)], zsem).wait()
    plsc.subcore_barrier()

    @pl.loop(0, NCHUNK, step=NB)
    def _(j):
        for b in range(NB):
            k = j + b
            pltpu.make_async_copy(y_hbm.at[sidxr.at[b]], rows[b],
                                  gsem[b]).wait()
            pltpu.make_async_copy(src_hbm.at[pl.ds(0, CH)], didxr.at[b],
                                  idsem[b]).wait()
            pltpu.async_copy(rows[b], acc.at[didxr.at[b]], ssem[b], add=True)
            pltpu.make_async_copy(rows[b], acc.at[didxr.at[b]],
                                  ssem[b]).wait()

            @pl.when(k + NB < NCHUNK)
            def _():
                fire_idx(k + NB, b)

            @pl.when(k + 3 < NCHUNK)
            def _():
                fire_gather((b + 3) % NB)
    plsc.subcore_barrier()

    pltpu.sync_copy(acc.at[pl.ds(r0, ROWS_PER_SUB)],
                    out_hbm.at[c, pl.ds(r0, ROWS_PER_SUB)])


_sc_edges = functools.partial(
    pl.kernel,
    out_type=jax.ShapeDtypeStruct((NC, NPAD, OUT_FEATS), jnp.float32),
    mesh=plsc.VectorSubcoreMesh(core_axis_name="c", subcore_axis_name="s"),
    scratch_types=[
        pltpu.VMEM((NB, CH), jnp.int32),
        pltpu.VMEM((NB, CH), jnp.int32),
        pltpu.VMEM((CH, OUT_FEATS), jnp.float32),
        pltpu.VMEM((CH, OUT_FEATS), jnp.float32),
        pltpu.VMEM((CH, OUT_FEATS), jnp.float32),
        pltpu.VMEM((CH, OUT_FEATS), jnp.float32),
        pltpu.VMEM((CH, OUT_FEATS), jnp.float32),
        pltpu.VMEM((ZROWS, OUT_FEATS), jnp.float32),
        pltpu.VMEM_SHARED((NPAD, OUT_FEATS), jnp.float32),
        pltpu.SemaphoreType.DMA((4 * NB + 1,)),
    ],
)(_sc_body)


def _combine_body(p_ref, b_ref, h_ref):
    h_ref[...] = p_ref[0] + p_ref[1] + b_ref[...]


def _tc_combine(p, bias2d):
    blk = 2000
    grid = N_NODES // blk
    return pl.pallas_call(
        _combine_body,
        grid=(grid,),
        in_specs=[
            pl.BlockSpec((NC, blk, OUT_FEATS), lambda i: (0, i, 0)),
            pl.BlockSpec((1, OUT_FEATS), lambda i: (0, 0)),
        ],
        out_specs=pl.BlockSpec((blk, OUT_FEATS), lambda i: (i, 0)),
        out_shape=jax.ShapeDtypeStruct((N_NODES, OUT_FEATS), jnp.float32),
    )(p, bias2d)


def kernel(x, edge_index, coeffs, bias):
    cw = jnp.transpose(coeffs, (2, 1, 0))
    y, sflat, dflat = _tc_poly(x, cw, edge_index)
    p = _sc_edges(y, sflat, dflat)
    return _tc_combine(p, bias.reshape(1, OUT_FEATS))


# speedup vs baseline: 1.0759x; 1.0759x over previous
import functools

import jax
import jax.numpy as jnp
from jax import lax
from jax.experimental import pallas as pl
from jax.experimental.pallas import tpu as pltpu
from jax.experimental.pallas import tpu_sc as plsc

N_NODES = 10000
IN_FEATS = 128
OUT_FEATS = 128
N_EDGES = 320000

NC = 2
NS = 16
CH = 128
EPW = N_EDGES // (NC * NS)
NCHUNK = EPW // CH
TAIL = EPW - NCHUNK * CH
NPAD = 10240
ROWS_PER_SUB = NPAD // NS
ZROWS = 32
IR = 6


def _poly_body(x_ref, c_ref, e_ref, y_ref, s_ref, d_ref):
    x = x_ref[...]
    dn = (((1,), (0,)), ((), ()))
    y = jnp.sum(c_ref[0], axis=0)[None, :]
    y = y + lax.dot_general(x, c_ref[1], dn, preferred_element_type=jnp.float32)
    x2 = x * x
    y = y + lax.dot_general(x2, c_ref[2], dn, preferred_element_type=jnp.float32)
    y = y + lax.dot_general(x2 * x, c_ref[3], dn, preferred_element_type=jnp.float32)
    y_ref[...] = y
    e = e_ref[...]
    eb = e.shape[1]
    i = pl.program_id(0)
    s_ref[pl.ds(i * eb, eb)] = e[0]
    d_ref[pl.ds(i * eb, eb)] = e[1]


def _tc_poly(x, cw, edge_index):
    blk = 2000
    grid = N_NODES // blk
    eb = N_EDGES // grid
    return pl.pallas_call(
        _poly_body,
        grid=(grid,),
        in_specs=[
            pl.BlockSpec((blk, IN_FEATS), lambda i: (i, 0)),
            pl.BlockSpec((4, IN_FEATS, OUT_FEATS), lambda i: (0, 0, 0)),
            pl.BlockSpec((2, eb), lambda i: (0, i)),
        ],
        out_specs=[
            pl.BlockSpec((blk, OUT_FEATS), lambda i: (i, 0)),
            pl.BlockSpec((N_EDGES,), lambda i: (0,)),
            pl.BlockSpec((N_EDGES,), lambda i: (0,)),
        ],
        out_shape=[
            jax.ShapeDtypeStruct((N_NODES, OUT_FEATS), jnp.float32),
            jax.ShapeDtypeStruct((N_EDGES,), jnp.int32),
            jax.ShapeDtypeStruct((N_EDGES,), jnp.int32),
        ],
    )(x, cw, edge_index)


def _sc_body(y_hbm, src_hbm, dst_hbm, out_hbm,
             sidxr, didxr, sidxt, didxt, rows0, rows1, zbuf, acc, sems):
    c = lax.axis_index("c")
    s = lax.axis_index("s")
    w = c * NS + s
    base = w * EPW
    r0 = s * ROWS_PER_SUB
    rows = (rows0, rows1)
    gsem = (sems.at[0], sems.at[1])
    ssem = (sems.at[2], sems.at[3])
    isem = tuple(sems.at[4 + t] for t in range(IR))
    idsem = tuple(sems.at[4 + IR + t] for t in range(IR))
    zsem = sems.at[4 + 2 * IR]

    def fire_idx(k, slot):
        off = base + k * CH
        pltpu.async_copy(src_hbm.at[pl.ds(off, CH)], sidxr.at[slot],
                         isem[slot])
        pltpu.async_copy(dst_hbm.at[pl.ds(off, CH)], didxr.at[slot],
                         idsem[slot])

    def fire_gather(slot, b):
        pltpu.make_async_copy(src_hbm.at[pl.ds(0, CH)], sidxr.at[slot],
                              isem[slot]).wait()
        pltpu.async_copy(y_hbm.at[sidxr.at[slot]], rows[b], gsem[b])

    for t in range(IR):
        fire_idx(t, t)
    fire_gather(0, 0)
    fire_gather(1, 1)

    def zrow(r, carry):
        for q in range(OUT_FEATS // 16):
            zbuf[r, pl.ds(q * 16, 16)] = jnp.zeros((16,), jnp.float32)
        return carry
    lax.fori_loop(0, ZROWS, zrow, 0)
    for t in range(ROWS_PER_SUB // ZROWS):
        pltpu.async_copy(zbuf, acc.at[pl.ds(r0 + t * ZROWS, ZROWS)], zsem)
    for t in range(ROWS_PER_SUB // ZROWS):
        pltpu.make_async_copy(zbuf, acc.at[pl.ds(r0, ZROWS)], zsem).wait()
    plsc.subcore_barrier()

    @pl.loop(0, NCHUNK, step=IR)
    def _(j):
        for b in range(IR):
            k = j + b
            buf = b % 2
            pltpu.make_async_copy(y_hbm.at[sidxr.at[b]], rows[buf],
                                  gsem[buf]).wait()
            pltpu.make_async_copy(src_hbm.at[pl.ds(0, CH)], didxr.at[b],
                                  idsem[b]).wait()
            pltpu.async_copy(rows[buf], acc.at[didxr.at[b]], ssem[buf],
                             add=True)
            pltpu.make_async_copy(rows[buf], acc.at[didxr.at[b]],
                                  ssem[buf]).wait()

            @pl.when(k + IR < NCHUNK)
            def _():
                fire_idx(k + IR, b)

            @pl.when(k + 2 < NCHUNK)
            def _():
                fire_gather((b + 2) % IR, buf)

    toff = base + NCHUNK * CH
    pltpu.sync_copy(src_hbm.at[pl.ds(toff, TAIL)], sidxt)
    pltpu.sync_copy(dst_hbm.at[pl.ds(toff, TAIL)], didxt.at[0])
    pltpu.async_copy(y_hbm.at[sidxt], rows0.at[pl.ds(0, TAIL)],
                     gsem[0]).wait()
    pltpu.sync_copy(rows0.at[pl.ds(0, TAIL)], acc.at[didxt.at[0]], add=True)
    plsc.subcore_barrier()

    pltpu.sync_copy(acc.at[pl.ds(r0, ROWS_PER_SUB)],
                    out_hbm.at[c, pl.ds(r0, ROWS_PER_SUB)])


_sc_edges = functools.partial(
    pl.kernel,
    out_type=jax.ShapeDtypeStruct((NC, NPAD, OUT_FEATS), jnp.float32),
    mesh=plsc.VectorSubcoreMesh(core_axis_name="c", subcore_axis_name="s"),
    scratch_types=[
        pltpu.VMEM((IR, CH), jnp.int32),
        pltpu.VMEM((IR, CH), jnp.int32),
        pltpu.VMEM((TAIL,), jnp.int32),
        pltpu.VMEM((1, TAIL), jnp.int32),
        pltpu.VMEM((CH, OUT_FEATS), jnp.float32),
        pltpu.VMEM((CH, OUT_FEATS), jnp.float32),
        pltpu.VMEM((ZROWS, OUT_FEATS), jnp.float32),
        pltpu.VMEM_SHARED((NPAD, OUT_FEATS), jnp.float32),
        pltpu.SemaphoreType.DMA((4 + 2 * IR + 1,)),
    ],
)(_sc_body)


def _combine_body(p_ref, b_ref, h_ref):
    h_ref[...] = p_ref[0] + p_ref[1] + b_ref[...]


def _tc_combine(p, bias2d):
    blk = 2000
    grid = N_NODES // blk
    return pl.pallas_call(
        _combine_body,
        grid=(grid,),
        in_specs=[
            pl.BlockSpec((NC, blk, OUT_FEATS), lambda i: (0, i, 0)),
            pl.BlockSpec((1, OUT_FEATS), lambda i: (0, 0)),
        ],
        out_specs=pl.BlockSpec((blk, OUT_FEATS), lambda i: (i, 0)),
        out_shape=jax.ShapeDtypeStruct((N_NODES, OUT_FEATS), jnp.float32),
    )(p, bias2d)


def kernel(x, edge_index, coeffs, bias):
    cw = jnp.transpose(coeffs, (2, 1, 0))
    y, sflat, dflat = _tc_poly(x, cw, edge_index)
    p = _sc_edges(y, sflat, dflat)
    return _tc_combine(p, bias.reshape(1, OUT_FEATS))


# speedup vs baseline: 1.1352x; 1.0551x over previous
import functools

import jax
import jax.numpy as jnp
from jax import lax
from jax.experimental import pallas as pl
from jax.experimental.pallas import tpu as pltpu
from jax.experimental.pallas import tpu_sc as plsc

N_NODES = 10000
IN_FEATS = 128
OUT_FEATS = 128
N_EDGES = 320000

NC = 2
NS = 16
CH = 80
EPW = N_EDGES // (NC * NS)
NCHUNK = EPW // CH
NPAD = 10240
ROWS_PER_SUB = NPAD // NS
ZROWS = 16
IR = 6
NRB = 3
MAIN = 120


def _poly_body(x_ref, c_ref, e_ref, y_ref, s_ref, d_ref):
    x = x_ref[...]
    dn = (((1,), (0,)), ((), ()))
    y = jnp.sum(c_ref[0], axis=0)[None, :]
    y = y + lax.dot_general(x, c_ref[1], dn, preferred_element_type=jnp.float32)
    x2 = x * x
    y = y + lax.dot_general(x2, c_ref[2], dn, preferred_element_type=jnp.float32)
    y = y + lax.dot_general(x2 * x, c_ref[3], dn, preferred_element_type=jnp.float32)
    y_ref[...] = y
    e = e_ref[...]
    eb = e.shape[1]
    i = pl.program_id(0)
    s_ref[pl.ds(i * eb, eb)] = e[0]
    d_ref[pl.ds(i * eb, eb)] = e[1]


def _tc_poly(x, cw, edge_index):
    blk = 2000
    grid = N_NODES // blk
    eb = N_EDGES // grid
    return pl.pallas_call(
        _poly_body,
        grid=(grid,),
        in_specs=[
            pl.BlockSpec((blk, IN_FEATS), lambda i: (i, 0)),
            pl.BlockSpec((4, IN_FEATS, OUT_FEATS), lambda i: (0, 0, 0)),
            pl.BlockSpec((2, eb), lambda i: (0, i)),
        ],
        out_specs=[
            pl.BlockSpec((blk, OUT_FEATS), lambda i: (i, 0)),
            pl.BlockSpec((N_EDGES,), lambda i: (0,)),
            pl.BlockSpec((N_EDGES,), lambda i: (0,)),
        ],
        out_shape=[
            jax.ShapeDtypeStruct((N_NODES, OUT_FEATS), jnp.float32),
            jax.ShapeDtypeStruct((N_EDGES,), jnp.int32),
            jax.ShapeDtypeStruct((N_EDGES,), jnp.int32),
        ],
    )(x, cw, edge_index)


def _sc_body(y_hbm, src_hbm, dst_hbm, out_hbm,
             sidxr, didxr, rows0, rows1, rows2, zbuf, acc, sems):
    c = lax.axis_index("c")
    s = lax.axis_index("s")
    w = c * NS + s
    base = w * EPW
    r0 = s * ROWS_PER_SUB
    rows = (rows0, rows1, rows2)
    gsem = tuple(sems.at[t] for t in range(NRB))
    ssem = tuple(sems.at[NRB + t] for t in range(NRB))
    isem = tuple(sems.at[2 * NRB + t] for t in range(IR))
    idsem = tuple(sems.at[2 * NRB + IR + t] for t in range(IR))
    zsem = sems.at[2 * NRB + 2 * IR]

    def fire_idx(k, slot):
        off = base + k * CH
        pltpu.async_copy(src_hbm.at[pl.ds(off, CH)], sidxr.at[slot],
                         isem[slot])
        pltpu.async_copy(dst_hbm.at[pl.ds(off, CH)], didxr.at[slot],
                         idsem[slot])

    def fire_gather(slot, b):
        pltpu.make_async_copy(src_hbm.at[pl.ds(0, CH)], sidxr.at[slot],
                              isem[slot]).wait()
        pltpu.async_copy(y_hbm.at[sidxr.at[slot]], rows[b], gsem[b])

    def wait_gather(b):
        pltpu.make_async_copy(y_hbm.at[sidxr.at[0]], rows[b], gsem[b]).wait()

    def fire_scatter(slot, b):
        pltpu.make_async_copy(src_hbm.at[pl.ds(0, CH)], didxr.at[slot],
                              idsem[slot]).wait()
        pltpu.async_copy(rows[b], acc.at[didxr.at[slot]], ssem[b], add=True)

    def wait_scatter(slot, b):
        pltpu.make_async_copy(rows[b], acc.at[didxr.at[slot]], ssem[b]).wait()

    for t in range(IR):
        fire_idx(t, t)
    fire_gather(0, 0)
    fire_gather(1, 1)

    def zrow(r, carry):
        for q in range(OUT_FEATS // 16):
            zbuf[r, pl.ds(q * 16, 16)] = jnp.zeros((16,), jnp.float32)
        return carry
    lax.fori_loop(0, ZROWS, zrow, 0)
    for t in range(ROWS_PER_SUB // ZROWS):
        pltpu.async_copy(zbuf, acc.at[pl.ds(r0 + t * ZROWS, ZROWS)], zsem)
    for t in range(ROWS_PER_SUB // ZROWS):
        pltpu.make_async_copy(zbuf, acc.at[pl.ds(r0, ZROWS)], zsem).wait()
    plsc.subcore_barrier()

    @pl.loop(0, MAIN, step=IR)
    def _(j):
        for b in range(IR):
            k = j + b
            buf = b % NRB
            wait_gather(buf)
            fire_scatter(b, buf)

            if b == 0:
                @pl.when(k > 0)
                def _():
                    wait_scatter(IR - 1, NRB - 1)
            else:
                wait_scatter(b - 1, (b - 1) % NRB)

            @pl.when(k + 2 < NCHUNK)
            def _():
                fire_gather((b + 2) % IR, (b + 2) % NRB)
            @pl.when(k + IR < NCHUNK)
            def _():
                fire_idx(k + IR, b)

    for t in range(MAIN, NCHUNK):
        wait_gather(t % NRB)
        fire_scatter(t % IR, t % NRB)
        wait_scatter((t - 1) % IR, (t - 1) % NRB)
        if t + 2 < NCHUNK:
            fire_gather((t + 2) % IR, (t + 2) % NRB)
    wait_scatter((NCHUNK - 1) % IR, (NCHUNK - 1) % NRB)
    plsc.subcore_barrier()

    pltpu.sync_copy(acc.at[pl.ds(r0, ROWS_PER_SUB)],
                    out_hbm.at[c, pl.ds(r0, ROWS_PER_SUB)])


_sc_edges = functools.partial(
    pl.kernel,
    out_type=jax.ShapeDtypeStruct((NC, NPAD, OUT_FEATS), jnp.float32),
    mesh=plsc.VectorSubcoreMesh(core_axis_name="c", subcore_axis_name="s"),
    scratch_types=[
        pltpu.VMEM((IR, CH), jnp.int32),
        pltpu.VMEM((IR, CH), jnp.int32),
        pltpu.VMEM((CH, OUT_FEATS), jnp.float32),
        pltpu.VMEM((CH, OUT_FEATS), jnp.float32),
        pltpu.VMEM((CH, OUT_FEATS), jnp.float32),
        pltpu.VMEM((ZROWS, OUT_FEATS), jnp.float32),
        pltpu.VMEM_SHARED((NPAD, OUT_FEATS), jnp.float32),
        pltpu.SemaphoreType.DMA((2 * NRB + 2 * IR + 1,)),
    ],
)(_sc_body)


def _combine_body(p_ref, b_ref, h_ref):
    h_ref[...] = p_ref[0] + p_ref[1] + b_ref[...]


def _tc_combine(p, bias2d):
    blk = 2000
    grid = N_NODES // blk
    return pl.pallas_call(
        _combine_body,
        grid=(grid,),
        in_specs=[
            pl.BlockSpec((NC, blk, OUT_FEATS), lambda i: (0, i, 0)),
            pl.BlockSpec((1, OUT_FEATS), lambda i: (0, 0)),
        ],
        out_specs=pl.BlockSpec((blk, OUT_FEATS), lambda i: (i, 0)),
        out_shape=jax.ShapeDtypeStruct((N_NODES, OUT_FEATS), jnp.float32),
    )(p, bias2d)


def kernel(x, edge_index, coeffs, bias):
    cw = jnp.transpose(coeffs, (2, 1, 0))
    y, sflat, dflat = _tc_poly(x, cw, edge_index)
    p = _sc_edges(y, sflat, dflat)
    return _tc_combine(p, bias.reshape(1, OUT_FEATS))
